# baseline (speedup 1.0000x reference)
import functools

import jax
import jax.numpy as jnp
from jax import lax
from jax.experimental import pallas as pl
from jax.experimental.pallas import tpu as pltpu
from jax.experimental.pallas import tpu_sc as plsc

_N = 16384
_D = 128
_T = 1000
_LANES = 16
_NW = 32
_CHUNK = _N // _NW
_ROWS = 128
_NCH = _CHUNK // _ROWS


def _sc_qsample(x, noise, tab1, tab2, t):
    mesh = plsc.VectorSubcoreMesh(core_axis_name="c", subcore_axis_name="s")

    @functools.partial(
        pl.kernel,
        mesh=mesh,
        out_type=jax.ShapeDtypeStruct((_N, _D), jnp.float32),
        scratch_types=[
            pltpu.VMEM((_CHUNK,), jnp.int32),
            pltpu.VMEM((_T + _LANES,), jnp.float32),
            pltpu.VMEM((_T + _LANES,), jnp.float32),
            [pltpu.VMEM((_ROWS, _D), jnp.float32)] * 2,
            [pltpu.VMEM((_ROWS, _D), jnp.float32)] * 2,
            [pltpu.VMEM((_ROWS, _D), jnp.float32)] * 2,
            [pltpu.SemaphoreType.DMA] * 2,
            [pltpu.SemaphoreType.DMA] * 2,
            [pltpu.SemaphoreType.DMA] * 2,
        ],
        compiler_params=pltpu.CompilerParams(use_tc_tiling_on_sc=False),
    )
    def qsample_kernel(x_hbm, n_hbm, tab1_hbm, tab2_hbm, t_hbm, o_hbm,
                       idx_v, t1_v, t2_v, xbufs, nbufs, obufs,
                       sxs, sns, sos):
        wid = lax.axis_index("s") * 2 + lax.axis_index("c")
        base = wid * _CHUNK

        def start_in(ch):
            b = ch % 2
            cx = pltpu.async_copy(
                x_hbm.at[pl.ds(base + ch * _ROWS, _ROWS)], xbufs[b], sxs[b])
            cn = pltpu.async_copy(
                n_hbm.at[pl.ds(base + ch * _ROWS, _ROWS)], nbufs[b], sns[b])
            return cx, cn

        in_flight = [start_in(0)]
        pltpu.sync_copy(t_hbm.at[pl.ds(base, _CHUNK)], idx_v)
        pltpu.sync_copy(tab1_hbm, t1_v.at[pl.ds(0, _T)])
        pltpu.sync_copy(tab2_hbm, t2_v.at[pl.ds(0, _T)])

        out_flight = [None, None]
        for ch in range(_NCH):
            b = ch % 2
            if ch + 1 < _NCH:
                in_flight.append(start_in(ch + 1))
            cx, cn = in_flight[ch]
            cx.wait()
            cn.wait()
            if out_flight[b] is not None:
                out_flight[b].wait()

            def group(g, carry, ch=ch, b=b):
                rb = g * _LANES
                idxv = idx_v[pl.ds(ch * _ROWS + rb, _LANES)]
                cs = []
                for i in range(_LANES):
                    ti = idxv[i]
                    cs.append((t1_v[pl.ds(ti, _LANES)][0],
                               t2_v[pl.ds(ti, _LANES)][0]))

                def jloop(j, carry2, rb=rb, b=b, cs=cs):
                    sl = pl.ds(j * _LANES, _LANES)
                    for i in range(_LANES):
                        r = rb + i
                        obufs[b][r, sl] = (cs[i][0] * xbufs[b][r, sl]
                                           + cs[i][1] * nbufs[b][r, sl])
                    return carry2

                lax.fori_loop(0, _D // _LANES, jloop, 0)
                return carry

            lax.fori_loop(0, _ROWS // _LANES, group, 0)
            out_flight[b] = pltpu.async_copy(
                obufs[b], o_hbm.at[pl.ds(base + ch * _ROWS, _ROWS)], sos[b])
        for cp in out_flight:
            if cp is not None:
                cp.wait()

    return qsample_kernel(x, noise, tab1, tab2, t)


def kernel(x, noise, sqrt_alphas_cumprod, sqrt_one_minus_alphas_cumprod, t):
    return _sc_qsample(x, noise, sqrt_alphas_cumprod,
                       sqrt_one_minus_alphas_cumprod, t.astype(jnp.int32))


# speedup vs baseline: 1.0293x; 1.0293x over previous
import functools

import jax
import jax.numpy as jnp
from jax import lax
from jax.experimental import pallas as pl
from jax.experimental.pallas import tpu as pltpu
from jax.experimental.pallas import tpu_sc as plsc

_N = 16384
_D = 128
_T = 1000
_LANES = 16
_NW = 32
_CHUNK = _N // _NW
_ROWS = 128
_NCH = _CHUNK // _ROWS


def _sc_qsample(x, noise, tab1, tab2, t):
    mesh = plsc.VectorSubcoreMesh(core_axis_name="c", subcore_axis_name="s")

    @functools.partial(
        pl.kernel,
        mesh=mesh,
        out_type=jax.ShapeDtypeStruct((_N, _D), jnp.float32),
        scratch_types=[
            pltpu.VMEM((_CHUNK,), jnp.int32),
            pltpu.VMEM((_CHUNK + _LANES,), jnp.float32),
            pltpu.VMEM((_CHUNK + _LANES,), jnp.float32),
            [pltpu.VMEM((_ROWS, _D), jnp.float32)] * 2,
            [pltpu.VMEM((_ROWS, _D), jnp.float32)] * 2,
            [pltpu.VMEM((_ROWS, _D), jnp.float32)] * 2,
            [pltpu.SemaphoreType.DMA] * 2,
            [pltpu.SemaphoreType.DMA] * 2,
            [pltpu.SemaphoreType.DMA] * 2,
        ],
        compiler_params=pltpu.CompilerParams(use_tc_tiling_on_sc=False),
    )
    def qsample_kernel(x_hbm, n_hbm, tab1_hbm, tab2_hbm, t_hbm, o_hbm,
                       idx_v, c1_v, c2_v, xbufs, nbufs, obufs,
                       sxs, sns, sos):
        wid = lax.axis_index("s") * 2 + lax.axis_index("c")
        base = wid * _CHUNK

        def start_in(ch):
            b = ch % 2
            cx = pltpu.async_copy(
                x_hbm.at[pl.ds(base + ch * _ROWS, _ROWS)], xbufs[b], sxs[b])
            cn = pltpu.async_copy(
                n_hbm.at[pl.ds(base + ch * _ROWS, _ROWS)], nbufs[b], sns[b])
            return cx, cn

        in_flight = [start_in(0)]
        pltpu.sync_copy(t_hbm.at[pl.ds(base, _CHUNK)], idx_v)
        cg1 = pltpu.async_copy(tab1_hbm.at[idx_v],
                               c1_v.at[pl.ds(0, _CHUNK)], sos[0])
        cg2 = pltpu.async_copy(tab2_hbm.at[idx_v],
                               c2_v.at[pl.ds(0, _CHUNK)], sos[1])
        cg1.wait()
        cg2.wait()

        out_flight = [None, None]
        for ch in range(_NCH):
            b = ch % 2
            if ch + 1 < _NCH:
                in_flight.append(start_in(ch + 1))
            cx, cn = in_flight[ch]
            cx.wait()
            cn.wait()
            if out_flight[b] is not None:
                out_flight[b].wait()

            def body(r, carry, ch=ch, b=b):
                c1 = c1_v[pl.ds(ch * _ROWS + r, _LANES)][0]
                c2 = c2_v[pl.ds(ch * _ROWS + r, _LANES)][0]
                for j in range(_D // _LANES):
                    sl = pl.ds(j * _LANES, _LANES)
                    obufs[b][r, sl] = (c1 * xbufs[b][r, sl]
                                       + c2 * nbufs[b][r, sl])
                return carry

            lax.fori_loop(0, _ROWS, body, 0)
            out_flight[b] = pltpu.async_copy(
                obufs[b], o_hbm.at[pl.ds(base + ch * _ROWS, _ROWS)], sos[b])
        for cp in out_flight:
            if cp is not None:
                cp.wait()

    return qsample_kernel(x, noise, tab1, tab2, t)


def kernel(x, noise, sqrt_alphas_cumprod, sqrt_one_minus_alphas_cumprod, t):
    return _sc_qsample(x, noise, sqrt_alphas_cumprod,
                       sqrt_one_minus_alphas_cumprod, t.astype(jnp.int32))


# speedup vs baseline: 1.0463x; 1.0165x over previous
import functools

import jax
import jax.numpy as jnp
from jax import lax
from jax.experimental import pallas as pl
from jax.experimental.pallas import tpu as pltpu
from jax.experimental.pallas import tpu_sc as plsc

_N = 16384
_D = 128
_T = 1000
_LANES = 16
_NW = 32
_CHUNK = _N // _NW
_ROWS = 128
_NCH = _CHUNK // _ROWS


def _sc_qsample(x, noise, tab1, tab2, t):
    mesh = plsc.VectorSubcoreMesh(core_axis_name="c", subcore_axis_name="s")

    @functools.partial(
        pl.kernel,
        mesh=mesh,
        out_type=jax.ShapeDtypeStruct((_N, _D), jnp.float32),
        scratch_types=[
            pltpu.VMEM((_CHUNK,), jnp.int32),
            pltpu.VMEM((_T + _LANES,), jnp.float32),
            pltpu.VMEM((_T + _LANES,), jnp.float32),
            pltpu.VMEM((2 * _ROWS, _D), jnp.float32),
            pltpu.VMEM((2 * _ROWS, _D), jnp.float32),
            pltpu.VMEM((2 * _ROWS, _D), jnp.float32),
            [pltpu.SemaphoreType.DMA] * 2,
            [pltpu.SemaphoreType.DMA] * 2,
            [pltpu.SemaphoreType.DMA] * 2,
        ],
        compiler_params=pltpu.CompilerParams(use_tc_tiling_on_sc=False),
    )
    def qsample_kernel(x_hbm, n_hbm, tab1_hbm, tab2_hbm, t_hbm, o_hbm,
                       idx_v, t1_v, t2_v, xbuf, nbuf, obuf,
                       sxs, sns, sos):
        wid = lax.axis_index("s") * 2 + lax.axis_index("c")
        base = wid * _CHUNK

        def start_in(ch, k):
            src = pl.ds(base + ch * _ROWS, _ROWS)
            dst = pl.ds(k * _ROWS, _ROWS)
            pltpu.async_copy(x_hbm.at[src], xbuf.at[dst], sxs[k])
            pltpu.async_copy(n_hbm.at[src], nbuf.at[dst], sns[k])

        def wait_in(k):
            dst = pl.ds(k * _ROWS, _ROWS)
            pltpu.make_async_copy(
                x_hbm.at[pl.ds(0, _ROWS)], xbuf.at[dst], sxs[k]).wait()
            pltpu.make_async_copy(
                n_hbm.at[pl.ds(0, _ROWS)], nbuf.at[dst], sns[k]).wait()

        def start_out(ch, k):
            pltpu.async_copy(obuf.at[pl.ds(k * _ROWS, _ROWS)],
                             o_hbm.at[pl.ds(base + ch * _ROWS, _ROWS)],
                             sos[k])

        def wait_out(k):
            pltpu.make_async_copy(
                o_hbm.at[pl.ds(0, _ROWS)],
                obuf.at[pl.ds(k * _ROWS, _ROWS)], sos[k]).wait()

        start_in(0, 0)
        pltpu.sync_copy(t_hbm.at[pl.ds(base, _CHUNK)], idx_v)
        pltpu.sync_copy(tab1_hbm, t1_v.at[pl.ds(0, _T)])
        pltpu.sync_copy(tab2_hbm, t2_v.at[pl.ds(0, _T)])
        start_in(1, 1)

        @pl.loop(0, _NCH)
        def chunk(ch):
            par = ch % 2

            for k in (0, 1):
                @pl.when(par == k)
                def _(k=k, ch=ch):
                    wait_in(k)

                @pl.when(jnp.logical_and(par == k, ch >= 2))
                def _(k=k, ch=ch):
                    wait_out(k)

            half = par * _ROWS

            def body(g, carry, ch=ch, half=half):
                rbase = g * _LANES
                idxv = idx_v[pl.ds(ch * _ROWS + rbase, _LANES)]
                for i in range(_LANES):
                    ti = idxv[i]
                    c1 = t1_v[pl.ds(ti, _LANES)][0]
                    c2 = t2_v[pl.ds(ti, _LANES)][0]
                    r = half + rbase + i
                    for j in range(_D // _LANES):
                        sl = pl.ds(j * _LANES, _LANES)
                        obuf[r, sl] = (c1 * xbuf[r, sl]
                                       + c2 * nbuf[r, sl])
                return carry

            lax.fori_loop(0, _ROWS // _LANES, body, 0)

            for k in (0, 1):
                @pl.when(par == k)
                def _(k=k, ch=ch):
                    start_out(ch, k)

                @pl.when(jnp.logical_and(par == k, ch + 2 < _NCH))
                def _(k=k, ch=ch):
                    start_in(ch + 2, k)

        wait_out(0)
        wait_out(1)

    return qsample_kernel(x, noise, tab1, tab2, t)


def kernel(x, noise, sqrt_alphas_cumprod, sqrt_one_minus_alphas_cumprod, t):
    return _sc_qsample(x, noise, sqrt_alphas_cumprod,
                       sqrt_one_minus_alphas_cumprod, t.astype(jnp.int32))


# speedup vs baseline: 1.5898x; 1.5194x over previous
import functools

import jax
import jax.numpy as jnp
from jax import lax
from jax.experimental import pallas as pl
from jax.experimental.pallas import tpu as pltpu
from jax.experimental.pallas import tpu_sc as plsc

_N = 16384
_D = 128
_T = 1000
_LANES = 16
_NW = 32
_CHUNK = _N // _NW
_ROWS = 128
_NCH = _CHUNK // _ROWS


def _sc_qsample(x, noise, tab1, tab2, t):
    mesh = plsc.VectorSubcoreMesh(core_axis_name="c", subcore_axis_name="s")

    @functools.partial(
        pl.kernel,
        mesh=mesh,
        out_type=jax.ShapeDtypeStruct((_N, _D), jnp.float32),
        scratch_types=[
            pltpu.VMEM((_CHUNK,), jnp.int32),
            pltpu.VMEM((_T + _LANES,), jnp.float32),
            pltpu.VMEM((_T + _LANES,), jnp.float32),
            [pltpu.VMEM((_ROWS, _D), jnp.float32)] * 2,
            [pltpu.VMEM((_ROWS, _D), jnp.float32)] * 2,
            [pltpu.VMEM((_ROWS, _D), jnp.float32)] * 2,
            [pltpu.SemaphoreType.DMA] * 2,
            [pltpu.SemaphoreType.DMA] * 2,
            [pltpu.SemaphoreType.DMA] * 2,
        ],
        compiler_params=pltpu.CompilerParams(use_tc_tiling_on_sc=False),
    )
    def qsample_kernel(x_hbm, n_hbm, tab1_hbm, tab2_hbm, t_hbm, o_hbm,
                       idx_v, t1_v, t2_v, xbufs, nbufs, obufs,
                       sxs, sns, sos):
        wid = lax.axis_index("s") * 2 + lax.axis_index("c")
        base = wid * _CHUNK

        def start_in(ch):
            b = ch % 2
            cx = pltpu.async_copy(
                x_hbm.at[pl.ds(base + ch * _ROWS, _ROWS)], xbufs[b], sxs[b])
            cn = pltpu.async_copy(
                n_hbm.at[pl.ds(base + ch * _ROWS, _ROWS)], nbufs[b], sns[b])
            return cx, cn

        in_flight = [start_in(0)]
        pltpu.sync_copy(t_hbm.at[pl.ds(base, _CHUNK)], idx_v)
        pltpu.sync_copy(tab1_hbm, t1_v.at[pl.ds(0, _T)])
        pltpu.sync_copy(tab2_hbm, t2_v.at[pl.ds(0, _T)])

        out_flight = [None, None]
        for ch in range(_NCH):
            b = ch % 2
            if ch + 1 < _NCH:
                in_flight.append(start_in(ch + 1))
            cx, cn = in_flight[ch]
            cx.wait()
            cn.wait()
            if out_flight[b] is not None:
                out_flight[b].wait()

            def body(g, carry, ch=ch, b=b):
                rbase = g * _LANES
                idxv = idx_v[pl.ds(ch * _ROWS + rbase, _LANES)]
                for i in range(_LANES):
                    ti = idxv[i]
                    c1 = t1_v[pl.ds(ti, _LANES)][0]
                    c2 = t2_v[pl.ds(ti, _LANES)][0]
                    r = rbase + i
                    for j in range(_D // _LANES):
                        sl = pl.ds(j * _LANES, _LANES)
                        obufs[b][r, sl] = (c1 * xbufs[b][r, sl]
                                           + c2 * nbufs[b][r, sl])
                return carry

            lax.fori_loop(0, _ROWS // _LANES, body, 0)
            out_flight[b] = pltpu.async_copy(
                obufs[b], o_hbm.at[pl.ds(base + ch * _ROWS, _ROWS)], sos[b])
        for cp in out_flight:
            if cp is not None:
                cp.wait()

    return qsample_kernel(x, noise, tab1, tab2, t)


def kernel(x, noise, sqrt_alphas_cumprod, sqrt_one_minus_alphas_cumprod, t):
    return _sc_qsample(x, noise, sqrt_alphas_cumprod,
                       sqrt_one_minus_alphas_cumprod, t.astype(jnp.int32))
